# baseline (speedup 1.0000x reference)
import jax
import jax.numpy as jnp
from jax.experimental import pallas as pl


def _copy_block(x_ref, o_ref):
    o_ref[...] = x_ref[...]


def kernel(x, hippocampus, neocortex):
    B, S, H = x.shape
    x2 = x.reshape(B * S, H)
    rows = x2.shape[0]
    block_rows = 2048
    out = pl.pallas_call(
        _copy_block,
        out_shape=jax.ShapeDtypeStruct(x2.shape, x2.dtype),
        grid=(rows // block_rows,),
        in_specs=[pl.BlockSpec((block_rows, H), lambda i: (i, 0))],
        out_specs=pl.BlockSpec((block_rows, H), lambda i: (i, 0)),
    )(x2)
    return out.reshape(B, S, H)
